# baseline (speedup 1.0000x reference)
import functools

import jax
import jax.numpy as jnp
from jax import lax
from jax.experimental import pallas as pl
from jax.experimental.pallas import tpu as pltpu
from jax.experimental.pallas import tpu_sc as plsc

_CHUNK_ROWS = 32
_NBUF = 3


def kernel(x, embedding):
    seq_len = x.shape[1]
    d_model = embedding.shape[-1]
    total_rows = embedding.shape[1]

    info = plsc.get_sparse_core_info()
    num_cores = info.num_cores
    num_workers = num_cores * info.num_subcores
    rows_per_w = seq_len // num_workers
    chunk = min(_CHUNK_ROWS, rows_per_w)
    nch = rows_per_w // chunk
    nbuf = min(_NBUF, nch)

    mesh = plsc.VectorSubcoreMesh(core_axis_name="c", subcore_axis_name="s")

    @functools.partial(
        pl.kernel,
        out_type=jax.ShapeDtypeStruct((seq_len, d_model), embedding.dtype),
        mesh=mesh,
        scratch_types=[
            pltpu.VMEM((nbuf, chunk, d_model), embedding.dtype),
            pltpu.SemaphoreType.DMA((nbuf,)),
            pltpu.SemaphoreType.DMA((nbuf,)),
        ],
    )
    def copy_k(emb_hbm, out_hbm, buf, in_sems, out_sems):
        wid = lax.axis_index("s") * num_cores + lax.axis_index("c")
        base = wid * rows_per_w

        def in_copy(ch):
            return pltpu.make_async_copy(
                emb_hbm.at[pl.ds(base + ch * chunk, chunk), :],
                buf.at[ch % nbuf],
                in_sems.at[ch % nbuf],
            )

        def out_copy(ch):
            return pltpu.make_async_copy(
                buf.at[ch % nbuf],
                out_hbm.at[pl.ds(base + ch * chunk, chunk), :],
                out_sems.at[ch % nbuf],
            )

        for ch in range(nbuf):
            in_copy(ch).start()
        for ch in range(nch):
            in_copy(ch).wait()
            out_copy(ch).start()
            nxt = ch + nbuf
            if nxt < nch:
                out_copy(ch).wait()
                in_copy(nxt).start()
        for ch in range(max(nch - nbuf, 0), nch):
            out_copy(ch).wait()

    out2d = copy_k(embedding.reshape(total_rows, d_model))
    return out2d.reshape(1, seq_len, d_model)


# speedup vs baseline: 2.8411x; 2.8411x over previous
import jax
import jax.numpy as jnp
from jax.experimental import pallas as pl
from jax.experimental.pallas import tpu as pltpu

_CHUNKS = 2


def kernel(x, embedding):
    seq_len = x.shape[1]
    d_model = embedding.shape[-1]
    chunks = _CHUNKS
    while seq_len % chunks != 0 and chunks > 1:
        chunks //= 2
    rows = seq_len // chunks

    def body(emb_hbm, out_hbm, vmem, in_sems, out_sems):
        for k in range(chunks):
            pltpu.make_async_copy(
                emb_hbm.at[0, pl.ds(k * rows, rows), :], vmem.at[k], in_sems.at[k]
            ).start()
        for k in range(chunks):
            pltpu.make_async_copy(
                emb_hbm.at[0, pl.ds(k * rows, rows), :], vmem.at[k], in_sems.at[k]
            ).wait()
            pltpu.make_async_copy(
                vmem.at[k], out_hbm.at[0, pl.ds(k * rows, rows), :], out_sems.at[k]
            ).start()
        for k in range(chunks):
            pltpu.make_async_copy(
                vmem.at[k], out_hbm.at[0, pl.ds(k * rows, rows), :], out_sems.at[k]
            ).wait()

    return pl.pallas_call(
        body,
        in_specs=[pl.BlockSpec(memory_space=pl.ANY)],
        out_specs=pl.BlockSpec(memory_space=pl.ANY),
        out_shape=jax.ShapeDtypeStruct((1, seq_len, d_model), embedding.dtype),
        scratch_shapes=[
            pltpu.VMEM((chunks, rows, d_model), embedding.dtype),
            pltpu.SemaphoreType.DMA((chunks,)),
            pltpu.SemaphoreType.DMA((chunks,)),
        ],
    )(embedding)
